# baseline (speedup 1.0000x reference)
import functools

import jax
import jax.numpy as jnp
from jax import lax
from jax.experimental import pallas as pl
from jax.experimental.pallas import tpu as pltpu
from jax.experimental.pallas import tpu_sc as plsc

_NC = 2
_NS = 16
_NW = _NC * _NS
_L = 16
_BT = 128
_DT = 8


def _body(seq, dim, table_hbm, idx_hbm, out_hbm, idx_v, g_v, t_v, gsem, osem):
    wid = lax.axis_index("s") * _NC + lax.axis_index("c")
    n_dt = dim // _DT
    pltpu.sync_copy(idx_hbm.at[:, wid], idx_v)

    def gather_desc(s, p):
        return pltpu.make_async_copy(
            table_hbm.at[idx_v.at[s // _DT, lax.rem(s, _DT)]],
            g_v.at[p],
            gsem.at[p],
        )

    def write_desc(s, p):
        return pltpu.make_async_copy(
            t_v.at[p],
            out_hbm.at[s, :, wid],
            osem.at[p],
        )

    rows = [lax.iota(jnp.int32, _L) + c0 for c0 in range(0, _BT, _L)]
    pvs = [jnp.full((_L,), p, jnp.int32) for p in (0, 1)]

    gather_desc(0, 0).start()

    @pl.loop(0, seq, step=2)
    def _(s0):
        for h in (0, 1):
            s = s0 + h
            p = h
            pn = 1 - h

            gather_desc(s, p).wait()

            @pl.when(s + 1 < seq)
            def _():
                gather_desc(s + 1, pn).start()

            @pl.when(s >= 2)
            def _():
                write_desc(s - 2, p).wait()

            @pl.loop(0, n_dt)
            def _(i):
                for r in range(_DT):
                    col = jnp.full((_L,), i * _DT + r, jnp.int32)
                    for j, rowv in enumerate(rows):
                        v = plsc.load_gather(g_v, [pvs[p], rowv, col])
                        t_v[p, i, r, pl.ds(j * _L, _L)] = v

            write_desc(s, p).start()

    write_desc(seq - 2, 0).wait()
    write_desc(seq - 1, 1).wait()


@functools.partial(jax.jit, static_argnames=("batch", "seq", "dim"))
def _gather(table, idx4, *, batch, seq, dim):
    mesh = plsc.VectorSubcoreMesh(core_axis_name="c", subcore_axis_name="s")
    run = pl.kernel(
        functools.partial(_body, seq, dim),
        out_type=jax.ShapeDtypeStruct(
            (seq, dim // _DT, batch // _BT, _DT, _BT), jnp.float32
        ),
        mesh=mesh,
        scratch_types=[
            pltpu.VMEM((seq // _DT, _DT, _BT), jnp.int32),
            pltpu.VMEM((2, _BT, dim), jnp.float32),
            pltpu.VMEM((2, dim // _DT, _DT, _BT), jnp.float32),
            pltpu.SemaphoreType.DMA((2,)),
            pltpu.SemaphoreType.DMA((2,)),
        ],
        compiler_params=pltpu.CompilerParams(
            use_tc_tiling_on_sc=False, needs_layout_passes=False
        ),
    )
    return run(table, idx4)


def kernel(gene_ids, table):
    batch, seq = gene_ids.shape
    dim = table.shape[1]
    assert batch == _NW * _BT
    assert seq % _DT == 0 and seq % 2 == 0 and dim % _DT == 0
    idx4 = jnp.transpose(
        gene_ids.T.reshape(seq // _DT, _DT, batch // _BT, _BT), (0, 2, 1, 3)
    )
    x = _gather(table, idx4, batch=batch, seq=seq, dim=dim)
    return jnp.transpose(x, (2, 4, 0, 1, 3)).reshape(batch, seq, dim)


# speedup vs baseline: 2.8329x; 2.8329x over previous
import functools

import jax
import jax.numpy as jnp
from jax import lax
from jax.experimental import pallas as pl
from jax.experimental.pallas import tpu as pltpu
from jax.experimental.pallas import tpu_sc as plsc

_NC = 2
_NS = 16
_NW = _NC * _NS
_L = 16
_BT = 128
_DT = 8


def _body(seq, dim, table_hbm, idx_hbm, out_hbm, idx_v, g_v, t_v, gsem, osem):
    wid = lax.axis_index("s") * _NC + lax.axis_index("c")
    n_dt = dim // _DT
    pltpu.sync_copy(idx_hbm.at[:, wid], idx_v)

    def gather_desc(s, p):
        return pltpu.make_async_copy(
            table_hbm.at[idx_v.at[s // _DT, lax.rem(s, _DT)]],
            g_v.at[p],
            gsem.at[p],
        )

    def write_desc(s, p):
        return pltpu.make_async_copy(
            t_v.at[p, :, :, pl.ds(0, _BT)],
            out_hbm.at[s, :, wid],
            osem.at[p],
        )

    dsplit = []
    for d0 in range(0, dim, _L):
        dvec = lax.iota(jnp.int32, _L) + d0
        dsplit.append((dvec // _DT, lax.rem(dvec, _DT)))

    gather_desc(0, 0).start()

    @pl.loop(0, seq, step=2)
    def _(s0):
        for h in (0, 1):
            s = s0 + h
            p = h
            pn = 1 - h

            gather_desc(s, p).wait()

            @pl.when(s + 1 < seq)
            def _():
                gather_desc(s + 1, pn).start()

            @pl.when(s >= 2)
            def _():
                write_desc(s - 2, p).wait()

            @pl.loop(0, _BT, unroll=4)
            def _(c):
                cv = jnp.full((_L,), c, jnp.int32)
                for k, (iv, rv) in enumerate(dsplit):
                    v = g_v[p, c, pl.ds(k * _L, _L)]
                    plsc.store_scatter(t_v.at[p], [iv, rv, cv], v)

            write_desc(s, p).start()

    write_desc(seq - 2, 0).wait()
    write_desc(seq - 1, 1).wait()


@functools.partial(jax.jit, static_argnames=("batch", "seq", "dim"))
def _gather(table, idx4, *, batch, seq, dim):
    mesh = plsc.VectorSubcoreMesh(core_axis_name="c", subcore_axis_name="s")
    run = pl.kernel(
        functools.partial(_body, seq, dim),
        out_type=jax.ShapeDtypeStruct(
            (seq, dim // _DT, batch // _BT, _DT, _BT), jnp.float32
        ),
        mesh=mesh,
        scratch_types=[
            pltpu.VMEM((seq // _DT, _DT, _BT), jnp.int32),
            pltpu.VMEM((2, _BT, dim), jnp.float32),
            pltpu.VMEM((2, dim // _DT, _DT, _BT + 1), jnp.float32),
            pltpu.SemaphoreType.DMA((2,)),
            pltpu.SemaphoreType.DMA((2,)),
        ],
        compiler_params=pltpu.CompilerParams(
            use_tc_tiling_on_sc=False, needs_layout_passes=False
        ),
    )
    return run(table, idx4)


def kernel(gene_ids, table):
    batch, seq = gene_ids.shape
    dim = table.shape[1]
    assert batch == _NW * _BT
    assert seq % _DT == 0 and seq % 2 == 0 and dim % _DT == 0
    idx4 = jnp.transpose(
        gene_ids.T.reshape(seq // _DT, _DT, batch // _BT, _BT), (0, 2, 1, 3)
    )
    x = _gather(table, idx4, batch=batch, seq=seq, dim=dim)
    return jnp.transpose(x, (2, 4, 0, 1, 3)).reshape(batch, seq, dim)


# speedup vs baseline: 4.1818x; 1.4762x over previous
import functools

import jax
import jax.numpy as jnp
from jax import lax
from jax.experimental import pallas as pl
from jax.experimental.pallas import tpu as pltpu
from jax.experimental.pallas import tpu_sc as plsc

_NC = 2
_NS = 16
_NW = _NC * _NS
_L = 16
_BT = 128
_DT = 8


def _body(seq, dim, table_hbm, idx_hbm, out_hbm, idx_v, g_v, t_v, gsem, osem):
    wid = lax.axis_index("s") * _NC + lax.axis_index("c")
    n_dt = dim // _DT
    pltpu.sync_copy(idx_hbm.at[:, wid], idx_v)

    def gather_desc(s, p):
        return pltpu.make_async_copy(
            table_hbm.at[idx_v.at[s // _DT, lax.rem(s, _DT)]],
            g_v.at[p],
            gsem.at[p],
        )

    def write_desc(s, p):
        return pltpu.make_async_copy(
            t_v.at[p, :, :, pl.ds(0, _BT)],
            out_hbm.at[s, :, wid],
            osem.at[p],
        )

    dsplit = []
    for d0 in range(0, dim, _L):
        dvec = lax.iota(jnp.int32, _L) + d0
        dsplit.append((dvec // _DT, lax.rem(dvec, _DT)))

    gather_desc(0, 0).start()

    @pl.loop(0, seq, step=2)
    def _(s0):
        for h in (0, 1):
            s = s0 + h
            p = h
            pn = 1 - h

            gather_desc(s, p).wait()

            @pl.when(s + 1 < seq)
            def _():
                gather_desc(s + 1, pn).start()

            @pl.when(s >= 2)
            def _():
                write_desc(s - 2, p).wait()

            @plsc.parallel_loop(0, _BT, unroll=8)
            def _(c):
                cv = jnp.full((_L,), c, jnp.int32)
                for k, (iv, rv) in enumerate(dsplit):
                    v = g_v[p, c, pl.ds(k * _L, _L)]
                    plsc.store_scatter(t_v.at[p], [iv, rv, cv], v)

            write_desc(s, p).start()

    write_desc(seq - 2, 0).wait()
    write_desc(seq - 1, 1).wait()


@functools.partial(jax.jit, static_argnames=("batch", "seq", "dim"))
def _gather(table, idx4, *, batch, seq, dim):
    mesh = plsc.VectorSubcoreMesh(core_axis_name="c", subcore_axis_name="s")
    run = pl.kernel(
        functools.partial(_body, seq, dim),
        out_type=jax.ShapeDtypeStruct(
            (seq, dim // _DT, batch // _BT, _DT, _BT), jnp.float32
        ),
        mesh=mesh,
        scratch_types=[
            pltpu.VMEM((seq // _DT, _DT, _BT), jnp.int32),
            pltpu.VMEM((2, _BT, dim), jnp.float32),
            pltpu.VMEM((2, dim // _DT, _DT, _BT + 1), jnp.float32),
            pltpu.SemaphoreType.DMA((2,)),
            pltpu.SemaphoreType.DMA((2,)),
        ],
        compiler_params=pltpu.CompilerParams(
            use_tc_tiling_on_sc=False, needs_layout_passes=False
        ),
    )
    return run(table, idx4)


def kernel(gene_ids, table):
    batch, seq = gene_ids.shape
    dim = table.shape[1]
    assert batch == _NW * _BT
    assert seq % _DT == 0 and seq % 2 == 0 and dim % _DT == 0
    idx4 = jnp.transpose(
        gene_ids.T.reshape(seq // _DT, _DT, batch // _BT, _BT), (0, 2, 1, 3)
    )
    x = _gather(table, idx4, batch=batch, seq=seq, dim=dim)
    return jnp.transpose(x, (2, 4, 0, 1, 3)).reshape(batch, seq, dim)


# speedup vs baseline: 5.6468x; 1.3503x over previous
import functools

import jax
import jax.numpy as jnp
from jax import lax
from jax.experimental import pallas as pl
from jax.experimental.pallas import tpu as pltpu
from jax.experimental.pallas import tpu_sc as plsc

_NC = 2
_NS = 16
_NW = _NC * _NS
_L = 16
_BT = 128
_DT = 8


def _body(seq, dim, table_hbm, idx_hbm, out_hbm, idx_v, g_v, t_v, gsem, osem):
    wid = lax.axis_index("s") * _NC + lax.axis_index("c")
    n_dt = dim // _DT
    pltpu.sync_copy(idx_hbm.at[:, wid], idx_v)

    def gather_desc(s, p):
        return pltpu.make_async_copy(
            table_hbm.at[idx_v.at[s // _DT, lax.rem(s, _DT)]],
            g_v.at[p],
            gsem.at[p],
        )

    def write_desc(s, p):
        return pltpu.make_async_copy(
            t_v.at[p, :, :, pl.ds(0, _BT)],
            out_hbm.at[s, :, wid],
            osem.at[p],
        )

    dsplit = []
    for d0 in range(0, dim, _L):
        dvec = lax.iota(jnp.int32, _L) + d0
        dsplit.append((dvec // _DT, lax.rem(dvec, _DT)))

    gather_desc(0, 0).start()
    gather_desc(1, 1).start()

    @pl.loop(0, seq, step=4)
    def _(s0):
        for h in (0, 1, 2, 3):
            s = s0 + h
            p = h
            w = h % 2

            gather_desc(s, p).wait()

            @pl.when(s + 2 < seq)
            def _():
                gather_desc(s + 2, (h + 2) % 4).start()

            @pl.when(s >= 2)
            def _():
                write_desc(s - 2, w).wait()

            @plsc.parallel_loop(0, _BT, unroll=8)
            def _(c):
                cv = jnp.full((_L,), c, jnp.int32)
                for k, (iv, rv) in enumerate(dsplit):
                    v = g_v[p, c, pl.ds(k * _L, _L)]
                    plsc.store_scatter(t_v.at[w], [iv, rv, cv], v)

            write_desc(s, w).start()

    write_desc(seq - 2, 0).wait()
    write_desc(seq - 1, 1).wait()


@functools.partial(jax.jit, static_argnames=("batch", "seq", "dim"))
def _gather(table, idx4, *, batch, seq, dim):
    mesh = plsc.VectorSubcoreMesh(core_axis_name="c", subcore_axis_name="s")
    run = pl.kernel(
        functools.partial(_body, seq, dim),
        out_type=jax.ShapeDtypeStruct(
            (seq, dim // _DT, batch // _BT, _DT, _BT), jnp.float32
        ),
        mesh=mesh,
        scratch_types=[
            pltpu.VMEM((seq // _DT, _DT, _BT), jnp.int32),
            pltpu.VMEM((4, _BT, dim), jnp.float32),
            pltpu.VMEM((2, dim // _DT, _DT, _BT + 1), jnp.float32),
            pltpu.SemaphoreType.DMA((4,)),
            pltpu.SemaphoreType.DMA((2,)),
        ],
        compiler_params=pltpu.CompilerParams(
            use_tc_tiling_on_sc=False, needs_layout_passes=False
        ),
    )
    return run(table, idx4)


def kernel(gene_ids, table):
    batch, seq = gene_ids.shape
    dim = table.shape[1]
    assert batch == _NW * _BT
    assert seq % _DT == 0 and seq % 4 == 0 and dim % _DT == 0
    idx4 = jnp.transpose(
        gene_ids.T.reshape(seq // _DT, _DT, batch // _BT, _BT), (0, 2, 1, 3)
    )
    x = _gather(table, idx4, batch=batch, seq=seq, dim=dim)
    return jnp.transpose(x, (2, 4, 0, 1, 3)).reshape(batch, seq, dim)


# speedup vs baseline: 5.8865x; 1.0425x over previous
import functools

import jax
import jax.numpy as jnp
from jax import lax
from jax.experimental import pallas as pl
from jax.experimental.pallas import tpu as pltpu
from jax.experimental.pallas import tpu_sc as plsc

_NC = 2
_NS = 16
_NW = _NC * _NS
_L = 16
_BT = 128
_DT = 8


def _body(seq, dim, table_hbm, idx_hbm, out_hbm, idx_v, g_v, t_v, gsem, osem):
    wid = lax.axis_index("s") * _NC + lax.axis_index("c")
    n_dt = dim // _DT
    pltpu.sync_copy(idx_hbm.at[:, wid], idx_v)

    def gather_desc(s, p):
        return pltpu.make_async_copy(
            table_hbm.at[idx_v.at[s // _DT, lax.rem(s, _DT)]],
            g_v.at[p],
            gsem.at[p],
        )

    def write_desc(s, p):
        return pltpu.make_async_copy(
            t_v.at[p, :, :, pl.ds(0, _BT)],
            out_hbm.at[s, :, wid],
            osem.at[p],
        )

    dsplit = []
    for d0 in range(0, dim, _L):
        dvec = lax.iota(jnp.int32, _L) + d0
        dsplit.append((dvec // _DT, lax.rem(dvec, _DT)))

    gather_desc(0, 0).start()
    gather_desc(1, 1).start()
    gather_desc(2, 2).start()

    @pl.loop(0, seq, step=4)
    def _(s0):
        for h in (0, 1, 2, 3):
            s = s0 + h
            p = h
            w = h

            gather_desc(s, p).wait()

            @pl.when(s + 3 < seq)
            def _():
                gather_desc(s + 3, (h + 3) % 4).start()

            @pl.when(s >= 4)
            def _():
                write_desc(s - 4, w).wait()

            @plsc.parallel_loop(0, _BT, unroll=8)
            def _(c):
                cv = jnp.full((_L,), c, jnp.int32)
                for k, (iv, rv) in enumerate(dsplit):
                    v = g_v[p, c, pl.ds(k * _L, _L)]
                    plsc.store_scatter(t_v.at[w], [iv, rv, cv], v)

            write_desc(s, w).start()

    for e in range(4):
        write_desc(seq - 4 + e, e).wait()


@functools.partial(jax.jit, static_argnames=("batch", "seq", "dim"))
def _gather(table, idx4, *, batch, seq, dim):
    mesh = plsc.VectorSubcoreMesh(core_axis_name="c", subcore_axis_name="s")
    run = pl.kernel(
        functools.partial(_body, seq, dim),
        out_type=jax.ShapeDtypeStruct(
            (seq, dim // _DT, batch // _BT, _DT, _BT), jnp.float32
        ),
        mesh=mesh,
        scratch_types=[
            pltpu.VMEM((seq // _DT, _DT, _BT), jnp.int32),
            pltpu.VMEM((4, _BT, dim), jnp.float32),
            pltpu.VMEM((4, dim // _DT, _DT, _BT + 1), jnp.float32),
            pltpu.SemaphoreType.DMA((4,)),
            pltpu.SemaphoreType.DMA((4,)),
        ],
        compiler_params=pltpu.CompilerParams(
            use_tc_tiling_on_sc=False, needs_layout_passes=False
        ),
    )
    return run(table, idx4)


def kernel(gene_ids, table):
    batch, seq = gene_ids.shape
    dim = table.shape[1]
    assert batch == _NW * _BT
    assert seq % _DT == 0 and seq % 4 == 0 and dim % _DT == 0
    idx4 = jnp.transpose(
        gene_ids.T.reshape(seq // _DT, _DT, batch // _BT, _BT), (0, 2, 1, 3)
    )
    x = _gather(table, idx4, batch=batch, seq=seq, dim=dim)
    return jnp.transpose(x, (2, 4, 0, 1, 3)).reshape(batch, seq, dim)


# speedup vs baseline: 6.1230x; 1.0402x over previous
import functools

import jax
import jax.numpy as jnp
from jax import lax
from jax.experimental import pallas as pl
from jax.experimental.pallas import tpu as pltpu
from jax.experimental.pallas import tpu_sc as plsc

_NC = 2
_NS = 16
_NW = _NC * _NS
_L = 16
_BT = 128
_DT = 8


def _body(seq, dim, table_hbm, idx_hbm, out_hbm, idx_v, g_v, t_v, gsem, osem):
    wid = lax.axis_index("s") * _NC + lax.axis_index("c")
    n_dt = dim // _DT
    pltpu.sync_copy(idx_hbm.at[:, wid], idx_v)

    def gather_desc(s, p):
        return pltpu.make_async_copy(
            table_hbm.at[idx_v.at[s // _DT, lax.rem(s, _DT)]],
            g_v.at[p],
            gsem.at[p],
        )

    def write_desc(s, p):
        return pltpu.make_async_copy(
            t_v.at[p, :, :, pl.ds(0, _BT)],
            out_hbm.at[s, :, wid],
            osem.at[p],
        )

    dsplit = []
    for d0 in range(0, dim, _L):
        dvec = lax.iota(jnp.int32, _L) + d0
        dsplit.append((dvec // _DT, lax.rem(dvec, _DT)))

    gather_desc(0, 0).start()
    gather_desc(1, 1).start()
    gather_desc(2, 2).start()

    @pl.loop(0, seq, step=4)
    def _(s0):
        for h in (0, 1, 2, 3):
            s = s0 + h
            p = h
            w = h

            gather_desc(s, p).wait()

            @pl.when(s + 3 < seq)
            def _():
                gather_desc(s + 3, (h + 3) % 4).start()

            @pl.when(s >= 4)
            def _():
                write_desc(s - 4, w).wait()

            @plsc.parallel_loop(0, _BT, unroll=8)
            def _(c):
                cv = jnp.full((_L,), c, jnp.int32)
                for k, (iv, rv) in enumerate(dsplit):
                    v = g_v[p, c, pl.ds(k * _L, _L)]
                    plsc.store_scatter(t_v.at[w], [iv, rv, cv], v)

            write_desc(s, w).start()

    for e in range(4):
        write_desc(seq - 4 + e, e).wait()


@functools.partial(jax.jit, static_argnames=("batch", "seq", "dim"))
def _gather(table, idx4, *, batch, seq, dim):
    mesh = plsc.VectorSubcoreMesh(core_axis_name="c", subcore_axis_name="s")
    run = pl.kernel(
        functools.partial(_body, seq, dim),
        out_type=jax.ShapeDtypeStruct(
            (seq, dim // _DT, batch // _BT, _DT, _BT), jnp.float32
        ),
        mesh=mesh,
        scratch_types=[
            pltpu.VMEM((seq // _DT, _DT, _BT), jnp.int32),
            pltpu.VMEM((4, _BT, dim), jnp.float32),
            pltpu.VMEM((4, dim // _DT, _DT, _BT + 1), jnp.float32),
            pltpu.SemaphoreType.DMA((4,)),
            pltpu.SemaphoreType.DMA((4,)),
        ],
        compiler_params=pltpu.CompilerParams(
            use_tc_tiling_on_sc=False, needs_layout_passes=False
        ),
    )
    return run(table, idx4)


def kernel(gene_ids, table):
    batch, seq = gene_ids.shape
    dim = table.shape[1]
    assert batch == _NW * _BT
    assert seq % _DT == 0 and seq % 4 == 0 and dim % _DT == 0
    idx4 = 2 * jnp.transpose(
        gene_ids.T.reshape(seq // _DT, _DT, batch // _BT, _BT), (0, 2, 1, 3)
    )
    vpad = -table.shape[0] % _DT
    tab2 = jnp.pad(table, ((0, vpad), (0, dim))).reshape(-1, dim)
    x = _gather(tab2, idx4, batch=batch, seq=seq, dim=dim)
    return jnp.transpose(x, (2, 4, 0, 1, 3)).reshape(batch, seq, dim)
